# Initial kernel scaffold; baseline (speedup 1.0000x reference)
import jax
import jax.numpy as jnp
from jax.experimental import pallas as pl


def kernel(input, node2graph):
    raise NotImplementedError("write your pallas kernel here")


# speedup vs baseline: 1.4610x; 1.4610x over previous
import jax
import jax.numpy as jnp
from jax import lax
from jax.experimental import pallas as pl

_R, _C = 256, 128
_N = _R * _C
_LOG2N = 15


def _sort_body(x_ref, off_ref, out_x_ref, out_i_ref):
    x = x_ref[...]
    off = off_ref[...]
    rows = lax.broadcasted_iota(jnp.int32, (_R, _C), 0)
    cols = lax.broadcasted_iota(jnp.int32, (_R, _C), 1)
    ii = rows * _C + cols
    idx = ii

    for ks in range(1, _LOG2N + 1):
        k = 1 << ks
        for js in range(ks - 1, -1, -1):
            j = 1 << js
            mask_j = (ii & j) == 0
            up = (ii & k) == 0
            if j < _C:
                pv = jnp.where(mask_j, jnp.roll(x, -j, axis=1), jnp.roll(x, j, axis=1))
                pi = jnp.where(mask_j, jnp.roll(idx, -j, axis=1), jnp.roll(idx, j, axis=1))
            else:
                jr = j // _C
                pv = jnp.where(mask_j, jnp.roll(x, -jr, axis=0), jnp.roll(x, jr, axis=0))
                pi = jnp.where(mask_j, jnp.roll(idx, -jr, axis=0), jnp.roll(idx, jr, axis=0))
            lt = (x < pv) | ((x == pv) & (idx < pi))
            take_self = (mask_j == up) == lt
            x = jnp.where(take_self, x, pv)
            idx = jnp.where(take_self, idx, pi)

    out_x_ref[...] = x - off
    out_i_ref[...] = idx


def kernel(input, node2graph):
    step = jnp.max(input, axis=0) - jnp.min(input, axis=0) + 1.0
    offset = node2graph.astype(input.dtype) * step
    x = input + offset
    out_x, out_i = pl.pallas_call(
        _sort_body,
        out_shape=(
            jax.ShapeDtypeStruct((_R, _C), jnp.float32),
            jax.ShapeDtypeStruct((_R, _C), jnp.int32),
        ),
    )(x.reshape(_R, _C), offset.reshape(_R, _C))
    return out_x.reshape(_N), out_i.reshape(_N)
